# baseline (speedup 1.0000x reference)
import jax
import jax.numpy as jnp
from jax.experimental import pallas as pl
from jax.experimental.pallas import tpu as pltpu

_N = 1024
_TC = 256
_SB = 3


def _dist_kernel(shifts_ref, pos_ref, post_ref, dist_ref, mask_ref):
    s0 = pl.program_id(0) * _SB
    pxj = post_ref[0:1, :]
    pyj = post_ref[1:2, :]
    pzj = post_ref[2:3, :]
    for t in range(_SB):
        cxj = pxj + shifts_ref[s0 + t, 0]
        cyj = pyj + shifts_ref[s0 + t, 1]
        czj = pzj + shifts_ref[s0 + t, 2]
        for r in range(_N // _TC):
            rows = pl.ds(r * _TC, _TC)
            dx = cxj - pos_ref[rows, 0:1]
            dy = cyj - pos_ref[rows, 1:2]
            dz = czj - pos_ref[rows, 2:3]
            y = dx * dx + dy * dy + dz * dz + 1e-12
            dist_ref[t, rows, :] = y * jax.lax.rsqrt(y)
            mask_ref[t, rows, :] = (
                (y < 24.999998) & (y > 1.0000001e-12)
            ).astype(jnp.int8)


def kernel(positions, lattice, numbers):
    shifts_frac = jnp.asarray(
        [[i, j, k] for i in (-1, 0, 1) for j in (-1, 0, 1) for k in (-1, 0, 1)],
        dtype=jnp.float32,
    )
    shifts_cart = shifts_frac @ lattice
    pos = positions @ lattice
    post = pos.T

    n = pos.shape[0]

    dist, mask = pl.pallas_call(
        _dist_kernel,
        grid=(27 // _SB,),
        in_specs=[
            pl.BlockSpec(memory_space=pltpu.SMEM),
            pl.BlockSpec((n, 3), lambda s: (0, 0)),
            pl.BlockSpec((3, n), lambda s: (0, 0)),
        ],
        out_specs=[
            pl.BlockSpec((_SB, n, n), lambda s: (s, 0, 0)),
            pl.BlockSpec((_SB, n, n), lambda s: (s, 0, 0)),
        ],
        out_shape=[
            jax.ShapeDtypeStruct((27, n, n), jnp.float32),
            jax.ShapeDtypeStruct((27, n, n), jnp.int8),
        ],
    )(shifts_cart, pos, post)
    return dist, mask.astype(jnp.bool_)


# speedup vs baseline: 1.1291x; 1.1291x over previous
import jax
import jax.numpy as jnp
from jax.experimental import pallas as pl
from jax.experimental.pallas import tpu as pltpu

_N = 1024
_TC = 256


def _dist_kernel(shifts_ref, pos_ref, post_ref, dist_ref, mask_ref, g2_ref, a_ref):
    s = pl.program_id(0)
    pxj = post_ref[0:1, :]
    pyj = post_ref[1:2, :]
    pzj = post_ref[2:3, :]

    @pl.when(s == 0)
    def _build_gram():
        px = pos_ref[:, 0:1]
        py = pos_ref[:, 1:2]
        pz = pos_ref[:, 2:3]
        a_ref[:, :] = px * px + py * py + pz * pz
        for r in range(_N // _TC):
            rows = pl.ds(r * _TC, _TC)
            cxi = pos_ref[rows, 0:1]
            cyi = pos_ref[rows, 1:2]
            czi = pos_ref[rows, 2:3]
            g2_ref[rows, :] = 2.0 * (cxi * pxj + cyi * pyj + czi * pzj)

    sx = shifts_ref[s, 0]
    sy = shifts_ref[s, 1]
    sz = shifts_ref[s, 2]
    cxj = pxj + sx
    cyj = pyj + sy
    czj = pzj + sz
    b = cxj * cxj + cyj * cyj + czj * czj
    u = a_ref[:, :] - 2.0 * (pos_ref[:, 0:1] * sx
                             + pos_ref[:, 1:2] * sy
                             + pos_ref[:, 2:3] * sz)
    for r in range(_N // _TC):
        rows = pl.ds(r * _TC, _TC)
        y = jnp.maximum((u[r * _TC:(r + 1) * _TC, :] + b) - g2_ref[rows, :], 1e-12)
        dist_ref[0, rows, :] = y * jax.lax.rsqrt(y)
        mask_ref[0, rows, :] = (
            (y < 24.999998) & (y > 1.0000001e-12)
        ).astype(jnp.int8)

    @pl.when(s == 13)
    def _patch_diag():
        for r in range(_N // _TC):
            rows = pl.ds(r * _TC, _TC)
            row_ids = jax.lax.broadcasted_iota(jnp.int32, (_TC, _N), 0) + r * _TC
            col_ids = jax.lax.broadcasted_iota(jnp.int32, (_TC, _N), 1)
            eq = row_ids == col_ids
            dist_ref[0, rows, :] = jnp.where(eq, 1e-6, dist_ref[0, rows, :])
            mask_ref[0, rows, :] = jnp.where(
                eq, jnp.int8(0), mask_ref[0, rows, :])


def kernel(positions, lattice, numbers):
    shifts_frac = jnp.asarray(
        [[i, j, k] for i in (-1, 0, 1) for j in (-1, 0, 1) for k in (-1, 0, 1)],
        dtype=jnp.float32,
    )
    shifts_cart = shifts_frac @ lattice
    pos = positions @ lattice
    pos = pos - 0.5 * (lattice[0] + lattice[1] + lattice[2])
    post = pos.T

    n = pos.shape[0]

    dist, mask = pl.pallas_call(
        _dist_kernel,
        grid=(27,),
        in_specs=[
            pl.BlockSpec(memory_space=pltpu.SMEM),
            pl.BlockSpec((n, 3), lambda s: (0, 0)),
            pl.BlockSpec((3, n), lambda s: (0, 0)),
        ],
        out_specs=[
            pl.BlockSpec((1, n, n), lambda s: (s, 0, 0)),
            pl.BlockSpec((1, n, n), lambda s: (s, 0, 0)),
        ],
        out_shape=[
            jax.ShapeDtypeStruct((27, n, n), jnp.float32),
            jax.ShapeDtypeStruct((27, n, n), jnp.int8),
        ],
        scratch_shapes=[
            pltpu.VMEM((n, n), jnp.float32),
            pltpu.VMEM((n, 1), jnp.float32),
        ],
    )(shifts_cart, pos, post)
    return dist, mask.astype(jnp.bool_)


# speedup vs baseline: 1.1641x; 1.0310x over previous
import jax
import jax.numpy as jnp
from jax.experimental import pallas as pl
from jax.experimental.pallas import tpu as pltpu

_N = 1024
_TC = 256
_SB = 3


def _dist_kernel(shifts_ref, pos_ref, post_ref, dist_ref, mask_ref, g2_ref, a_ref):
    s = pl.program_id(0)
    pxj = post_ref[0:1, :]
    pyj = post_ref[1:2, :]
    pzj = post_ref[2:3, :]

    @pl.when(s == 0)
    def _build_gram():
        px = pos_ref[:, 0:1]
        py = pos_ref[:, 1:2]
        pz = pos_ref[:, 2:3]
        a_ref[:, :] = px * px + py * py + pz * pz
        for r in range(_N // _TC):
            rows = pl.ds(r * _TC, _TC)
            cxi = pos_ref[rows, 0:1]
            cyi = pos_ref[rows, 1:2]
            czi = pos_ref[rows, 2:3]
            g2_ref[rows, :] = 2.0 * (cxi * pxj + cyi * pyj + czi * pzj)

    for t in range(_SB):
        st = s * _SB + t
        sx = shifts_ref[st, 0]
        sy = shifts_ref[st, 1]
        sz = shifts_ref[st, 2]
        cxj = pxj + sx
        cyj = pyj + sy
        czj = pzj + sz
        b = cxj * cxj + cyj * cyj + czj * czj
        u = a_ref[:, :] - 2.0 * (pos_ref[:, 0:1] * sx
                                 + pos_ref[:, 1:2] * sy
                                 + pos_ref[:, 2:3] * sz)
        for r in range(_N // _TC):
            rows = pl.ds(r * _TC, _TC)
            y = jnp.maximum(
                (u[r * _TC:(r + 1) * _TC, :] + b) - g2_ref[rows, :], 1e-12)
            dist_ref[t, rows, :] = y * jax.lax.rsqrt(y)
            mask_ref[t, rows, :] = (
                (y < 24.999998) & (y > 1.0000001e-12)
            ).astype(jnp.int8)

        @pl.when(st == 13)
        def _patch_diag():
            for r in range(_N // _TC):
                rows = pl.ds(r * _TC, _TC)
                row_ids = jax.lax.broadcasted_iota(
                    jnp.int32, (_TC, _N), 0) + r * _TC
                col_ids = jax.lax.broadcasted_iota(jnp.int32, (_TC, _N), 1)
                eq = row_ids == col_ids
                dist_ref[t, rows, :] = jnp.where(eq, 1e-6, dist_ref[t, rows, :])
                mask_ref[t, rows, :] = jnp.where(
                    eq, jnp.int8(0), mask_ref[t, rows, :])


def kernel(positions, lattice, numbers):
    shifts_frac = jnp.asarray(
        [[i, j, k] for i in (-1, 0, 1) for j in (-1, 0, 1) for k in (-1, 0, 1)],
        dtype=jnp.float32,
    )
    shifts_cart = shifts_frac @ lattice
    pos = positions @ lattice
    pos = pos - 0.5 * (lattice[0] + lattice[1] + lattice[2])
    post = pos.T

    n = pos.shape[0]

    dist, mask = pl.pallas_call(
        _dist_kernel,
        grid=(27 // _SB,),
        in_specs=[
            pl.BlockSpec(memory_space=pltpu.SMEM),
            pl.BlockSpec((n, 3), lambda s: (0, 0)),
            pl.BlockSpec((3, n), lambda s: (0, 0)),
        ],
        out_specs=[
            pl.BlockSpec((_SB, n, n), lambda s: (s, 0, 0)),
            pl.BlockSpec((_SB, n, n), lambda s: (s, 0, 0)),
        ],
        out_shape=[
            jax.ShapeDtypeStruct((27, n, n), jnp.float32),
            jax.ShapeDtypeStruct((27, n, n), jnp.int8),
        ],
        scratch_shapes=[
            pltpu.VMEM((n, n), jnp.float32),
            pltpu.VMEM((n, 1), jnp.float32),
        ],
    )(shifts_cart, pos, post)
    return dist, mask.astype(jnp.bool_)
